# baseline (speedup 1.0000x reference)
import functools

import jax
import jax.numpy as jnp
from jax import lax
from jax.experimental import pallas as pl
from jax.experimental.pallas import tpu as pltpu
from jax.experimental.pallas import tpu_sc as plsc

I, C, H, W = 8, 28, 512, 512
HW = H * W
CP = 32
CE = 5
CO = C + CE
N = 64 * 64 * 64
EPS = 1e-8

NC, NS, L = 2, 16, 16
NW = NC * NS
PPW = N // NW
BLK = 512
NBLK = PPW // BLK
GCH = 128
NG = BLK // GCH
BN = 2048
IH = 4


def _tc_project_body(coord_ref, tr_ref, cam_ref, idx_ref, ext_ref):
    x = coord_ref[0:1, :]
    y = coord_ref[1:2, :]
    z = coord_ref[2:3, :]
    bf = lambda a: a.astype(jnp.bfloat16).astype(jnp.float32)
    xb, yb, zb = bf(x), bf(y), bf(z)
    for i in range(I):
        t = lambda k, l: tr_ref[i, k, l]
        p0 = (t(0, 0) * xb + t(0, 1) * yb) + (t(0, 2) * zb + t(0, 3))
        p1 = (t(1, 0) * xb + t(1, 1) * yb) + (t(1, 2) * zb + t(1, 3))
        d = (t(2, 0) * xb + t(2, 1) * yb) + (t(2, 2) * zb + t(2, 3))
        u = p0 / (d + EPS)
        v = p1 / (d + EPS)
        valid = (d > 0) & (u >= 0) & (u <= W - 1) & (v >= 0) & (v <= H - 1)
        validf = valid.astype(jnp.float32)
        ui = jnp.clip(jnp.round(u), 0, W - 1).astype(jnp.int32)
        vi = jnp.clip(jnp.round(v), 0, H - 1).astype(jnp.int32)
        ui = jnp.clip(ui, 0, W - 1)
        vi = jnp.clip(vi, 0, H - 1)
        pos = (jax.lax.broadcasted_iota(jnp.int32, (1, BN), 1)
               + pl.program_id(0) * BN)
        ih = (i % IH) * HW
        idx_ref[i:i + 1, :] = jnp.where(valid, ih + vi * W + ui, ih + pos)
        dx = x - cam_ref[i, 0]
        dy = y - cam_ref[i, 1]
        dz = z - cam_ref[i, 2]
        inv = 1.0 / (jnp.sqrt(dx * dx + dy * dy + dz * dz) + EPS)
        ext_ref[i, 0:1, :] = d
        ext_ref[i, 1:2, :] = validf
        ext_ref[i, 2:3, :] = dx * inv
        ext_ref[i, 3:4, :] = dy * inv
        ext_ref[i, 4:5, :] = dz * inv


_tc_project = pl.pallas_call(
    _tc_project_body,
    grid=(N // BN,),
    in_specs=[
        pl.BlockSpec((3, BN), lambda j: (0, j)),
        pl.BlockSpec(memory_space=pltpu.SMEM),
        pl.BlockSpec(memory_space=pltpu.SMEM),
    ],
    out_specs=[
        pl.BlockSpec((I, BN), lambda j: (0, j)),
        pl.BlockSpec((I, CE, BN), lambda j: (0, 0, j)),
    ],
    out_shape=[
        jax.ShapeDtypeStruct((I, N), jnp.int32),
        jax.ShapeDtypeStruct((I, CE, N), jnp.float32),
    ],
)


BH = 8


def _tc_table_body(img_ref, tab_ref):
    x = img_ref[0]
    x = x.reshape(C, BH * W)
    xp = jnp.concatenate([x, jnp.zeros((CP - C, BH * W), x.dtype)], axis=0)
    tab_ref[...] = xp.T


_tc_table = pl.pallas_call(
    _tc_table_body,
    grid=(IH, H // BH),
    in_specs=[pl.BlockSpec((1, C, BH, W), lambda i, h: (i, 0, h, 0))],
    out_specs=pl.BlockSpec((BH * W, CP), lambda i, h: (i * (H // BH) + h, 0)),
    out_shape=jax.ShapeDtypeStruct((IH * HW, CP), jnp.float32),
)


_sc_mesh = plsc.VectorSubcoreMesh(core_axis_name="c", subcore_axis_name="s",
                                  num_cores=NC, num_subcores=NS)


NBI = PPW // BLK
RPI = PPW // GCH
NPAIR = IH * NBI // 2


@functools.partial(
    pl.kernel,
    out_type=jax.ShapeDtypeStruct((IH, CO, N), jnp.float32),
    mesh=_sc_mesh,
    compiler_params=pltpu.CompilerParams(needs_layout_passes=False,
                                         use_tc_tiling_on_sc=False),
    scratch_types=[
        pltpu.VMEM((2, RPI, GCH), jnp.int32),
        pltpu.VMEM((BLK, CP), jnp.float32),
        pltpu.VMEM((BLK, CP), jnp.float32),
        pltpu.VMEM((CO, BLK), jnp.float32),
        pltpu.VMEM((CO, BLK), jnp.float32),
        pltpu.VMEM((CE, BLK), jnp.float32),
        pltpu.VMEM((CE, BLK), jnp.float32),
        pltpu.SemaphoreType.DMA,
        pltpu.SemaphoreType.DMA,
        pltpu.SemaphoreType.DMA,
        pltpu.SemaphoreType.DMA,
        pltpu.SemaphoreType.DMA,
        pltpu.SemaphoreType.DMA,
    ],
)
def _sc_gather(table_hbm, idx_hbm, ext_hbm, out_hbm, idxv,
               rows0, rows1, cb0, cb1, eb0, eb1,
               gs0, gs1, es0, es1, os0, os1):
    wid = lax.axis_index("s") * NC + lax.axis_index("c")
    iota = lax.iota(jnp.int32, L)
    rows = (rows0, rows1)
    cbs = (cb0, cb1)
    ebs = (eb0, eb1)
    gss = (gs0, gs1)
    ess = (es0, es1)
    oss = (os0, os1)

    def stage(ib):
        row0 = ib * (N // GCH) + wid * RPI
        pltpu.sync_copy(idx_hbm.at[pl.ds(row0, RPI)], idxv.at[ib % 2])

    def fire(g, p):
        ib = g // NBI
        lb = g - ib * NBI
        ipar = ib % 2
        for j in range(NG):
            pltpu.async_copy(table_hbm.at[idxv.at[ipar, lb * NG + j]],
                             rows[p].at[pl.ds(j * GCH, GCH)], gss[p])
        nb = wid * PPW + lb * BLK
        pltpu.async_copy(ext_hbm.at[ib, :, pl.ds(nb, BLK)], ebs[p], ess[p])

    def wait_fire(p):
        for j in range(NG):
            pltpu.make_async_copy(table_hbm.at[idxv.at[0, j]],
                                  rows[p].at[pl.ds(j * GCH, GCH)],
                                  gss[p]).wait()
        pltpu.make_async_copy(ext_hbm.at[0, :, pl.ds(0, BLK)], ebs[p],
                              ess[p]).wait()

    def wait_out(p):
        pltpu.make_async_copy(cbs[p], out_hbm.at[0, :, pl.ds(0, BLK)],
                              oss[p]).wait()

    def extract_and_out(g, p):
        ib = g // NBI
        lb = g - ib * NBI
        nb = wid * PPW + lb * BLK

        def grp(r, _):
            r0 = r * L
            ridx = r0 + iota
            vf = ebs[p][1, pl.ds(r0, L)]
            for c in range(C):
                cidx = jnp.full((L,), c, jnp.int32)
                val = plsc.load_gather(rows[p], [ridx, cidx])
                cbs[p][c, pl.ds(r0, L)] = val * vf
            for e in range(CE):
                cbs[p][C + e, pl.ds(r0, L)] = ebs[p][e, pl.ds(r0, L)]
            return 0

        lax.fori_loop(0, BLK // L, grp, 0)
        pltpu.async_copy(cbs[p], out_hbm.at[ib, :, pl.ds(nb, BLK)], oss[p])

    stage(0)
    fire(0, 0)

    def pair(p2, carry):
        gA = 2 * p2
        fire(gA + 1, 1)
        wait_fire(0)

        @pl.when(p2 >= 1)
        def _():
            wait_out(0)

        extract_and_out(gA, 0)

        @pl.when(jnp.logical_and((p2 + 1) % (NBI // 2) == 0, p2 < NPAIR - 1))
        def _():
            stage((p2 + 1) // (NBI // 2))

        @pl.when(p2 < NPAIR - 1)
        def _():
            fire(gA + 2, 0)

        wait_fire(1)

        @pl.when(p2 >= 1)
        def _():
            wait_out(1)

        extract_and_out(gA + 1, 1)
        return carry

    lax.fori_loop(0, NPAIR, pair, 0)
    wait_out(0)
    wait_out(1)


def kernel(images, transformations, T_cw, coordinates):
    coords2 = coordinates.reshape(3, N)
    R = T_cw[:, :3, :3]
    t = T_cw[:, :3, 3]
    cam = -jnp.einsum('ikl,ik->il', R, t)
    trb = transformations.astype(jnp.bfloat16).astype(jnp.float32)
    idx, ext = _tc_project(coords2, trb, cam)
    outs = []
    for h in range(I // IH):
        table = _tc_table(images[h * IH:(h + 1) * IH])
        idx_h = idx[h * IH:(h + 1) * IH].reshape(IH * N // GCH, GCH)
        outs.append(_sc_gather(table, idx_h, ext[h * IH:(h + 1) * IH]))
    out = jnp.concatenate(outs, axis=0)
    return out.reshape(I // 2, 2, CO, 64, 64, 64)


# speedup vs baseline: 2.1906x; 2.1906x over previous
import functools

import jax
import jax.numpy as jnp
from jax import lax
from jax.experimental import pallas as pl
from jax.experimental.pallas import tpu as pltpu
from jax.experimental.pallas import tpu_sc as plsc

I, C, H, W = 8, 28, 512, 512
HW = H * W
CP = 32
CE = 5
CO = C + CE
N = 64 * 64 * 64
EPS = 1e-8

NC, NS, L = 2, 16, 16
NW = NC * NS
PPW = N // NW
BLK = 512
NBLK = PPW // BLK
GCH = 64
NG = BLK // GCH
BN = 2048
IH = 4


def _tc_project_body(coord_ref, tr_ref, cam_ref, idx_ref, ext_ref):
    x = coord_ref[0:1, :]
    y = coord_ref[1:2, :]
    z = coord_ref[2:3, :]
    bf = lambda a: a.astype(jnp.bfloat16).astype(jnp.float32)
    xb, yb, zb = bf(x), bf(y), bf(z)
    for i in range(I):
        t = lambda k, l: tr_ref[i, k, l]
        p0 = (t(0, 0) * xb + t(0, 1) * yb) + (t(0, 2) * zb + t(0, 3))
        p1 = (t(1, 0) * xb + t(1, 1) * yb) + (t(1, 2) * zb + t(1, 3))
        d = (t(2, 0) * xb + t(2, 1) * yb) + (t(2, 2) * zb + t(2, 3))
        u = p0 / (d + EPS)
        v = p1 / (d + EPS)
        valid = (d > 0) & (u >= 0) & (u <= W - 1) & (v >= 0) & (v <= H - 1)
        validf = valid.astype(jnp.float32)
        ui = jnp.clip(jnp.round(u), 0, W - 1).astype(jnp.int32)
        vi = jnp.clip(jnp.round(v), 0, H - 1).astype(jnp.int32)
        ui = jnp.clip(ui, 0, W - 1)
        vi = jnp.clip(vi, 0, H - 1)
        pos = (jax.lax.broadcasted_iota(jnp.int32, (1, BN), 1)
               + pl.program_id(0) * BN)
        ih = i * HW
        idx_ref[i:i + 1, :] = jnp.where(valid, ih + vi * W + ui, ih + pos)
        dx = x - cam_ref[i, 0]
        dy = y - cam_ref[i, 1]
        dz = z - cam_ref[i, 2]
        inv = 1.0 / (jnp.sqrt(dx * dx + dy * dy + dz * dz) + EPS)
        ext_ref[i, 0:1, :] = d
        ext_ref[i, 1:2, :] = validf
        ext_ref[i, 2:3, :] = dx * inv
        ext_ref[i, 3:4, :] = dy * inv
        ext_ref[i, 4:5, :] = dz * inv


_tc_project = pl.pallas_call(
    _tc_project_body,
    grid=(N // BN,),
    in_specs=[
        pl.BlockSpec((3, BN), lambda j: (0, j)),
        pl.BlockSpec(memory_space=pltpu.SMEM),
        pl.BlockSpec(memory_space=pltpu.SMEM),
    ],
    out_specs=[
        pl.BlockSpec((I, BN), lambda j: (0, j)),
        pl.BlockSpec((I, CE, BN), lambda j: (0, 0, j)),
    ],
    out_shape=[
        jax.ShapeDtypeStruct((I, N), jnp.int32),
        jax.ShapeDtypeStruct((I, CE, N), jnp.float32),
    ],
)


BH = 8


def _tc_table_body(img_ref, tab_ref):
    x = img_ref[0]
    x = x.reshape(C, BH * W)
    xp = jnp.concatenate([x, jnp.zeros((CP - C, BH * W), x.dtype)], axis=0)
    tab_ref[...] = xp.T


_tc_table = pl.pallas_call(
    _tc_table_body,
    grid=(I, H // BH),
    in_specs=[pl.BlockSpec((1, C, BH, W), lambda i, h: (i, 0, h, 0))],
    out_specs=pl.BlockSpec((BH * W, CP), lambda i, h: (i * (H // BH) + h, 0)),
    out_shape=jax.ShapeDtypeStruct((I * HW, CP), jnp.float32),
)


_sc_mesh = plsc.VectorSubcoreMesh(core_axis_name="c", subcore_axis_name="s",
                                  num_cores=NC, num_subcores=NS)


NBI = PPW // BLK
RPI = PPW // GCH
NPAIR = I * NBI // 2


@functools.partial(
    pl.kernel,
    out_type=jax.ShapeDtypeStruct((I, CO, N), jnp.float32),
    mesh=_sc_mesh,
    compiler_params=pltpu.CompilerParams(needs_layout_passes=False,
                                         use_tc_tiling_on_sc=False),
    scratch_types=[
        pltpu.VMEM((2, RPI, GCH), jnp.int32),
        pltpu.VMEM((BLK, CP), jnp.float32),
        pltpu.VMEM((BLK, CP), jnp.float32),
        pltpu.VMEM((CO, BLK), jnp.float32),
        pltpu.VMEM((CO, BLK), jnp.float32),
        pltpu.VMEM((CE, BLK), jnp.float32),
        pltpu.VMEM((CE, BLK), jnp.float32),
        pltpu.SemaphoreType.DMA,
        pltpu.SemaphoreType.DMA,
        pltpu.SemaphoreType.DMA,
        pltpu.SemaphoreType.DMA,
        pltpu.SemaphoreType.DMA,
        pltpu.SemaphoreType.DMA,
    ],
)
def _sc_gather(table_hbm, idx_hbm, ext_hbm, out_hbm, idxv,
               rows0, rows1, cb0, cb1, eb0, eb1,
               gs0, gs1, es0, es1, os0, os1):
    wid = lax.axis_index("s") * NC + lax.axis_index("c")
    iota = lax.iota(jnp.int32, L)
    rows = (rows0, rows1)
    cbs = (cb0, cb1)
    ebs = (eb0, eb1)
    gss = (gs0, gs1)
    ess = (es0, es1)
    oss = (os0, os1)

    def stage(ib):
        row0 = ib * (N // GCH) + wid * RPI
        pltpu.sync_copy(idx_hbm.at[pl.ds(row0, RPI)], idxv.at[ib % 2])

    def fire(g, p):
        ib = g // NBI
        lb = g - ib * NBI
        ipar = ib % 2
        for j in range(NG):
            pltpu.async_copy(table_hbm.at[idxv.at[ipar, lb * NG + j]],
                             rows[p].at[pl.ds(j * GCH, GCH)], gss[p])
        nb = wid * PPW + lb * BLK
        pltpu.async_copy(ext_hbm.at[ib, :, pl.ds(nb, BLK)], ebs[p], ess[p])

    def wait_fire(p):
        for j in range(NG):
            pltpu.make_async_copy(table_hbm.at[idxv.at[0, j]],
                                  rows[p].at[pl.ds(j * GCH, GCH)],
                                  gss[p]).wait()
        pltpu.make_async_copy(ext_hbm.at[0, :, pl.ds(0, BLK)], ebs[p],
                              ess[p]).wait()

    def wait_out(p):
        pltpu.make_async_copy(cbs[p], out_hbm.at[0, :, pl.ds(0, BLK)],
                              oss[p]).wait()

    def extract_and_out(g, p):
        ib = g // NBI
        lb = g - ib * NBI
        nb = wid * PPW + lb * BLK

        def grp(r, _):
            r0 = r * L
            ridx = r0 + iota
            vf = ebs[p][1, pl.ds(r0, L)]
            for c in range(C):
                cidx = jnp.full((L,), c, jnp.int32)
                val = plsc.load_gather(rows[p], [ridx, cidx])
                cbs[p][c, pl.ds(r0, L)] = val * vf
            for e in range(CE):
                cbs[p][C + e, pl.ds(r0, L)] = ebs[p][e, pl.ds(r0, L)]
            return 0

        lax.fori_loop(0, BLK // L, grp, 0)
        pltpu.async_copy(cbs[p], out_hbm.at[ib, :, pl.ds(nb, BLK)], oss[p])

    stage(0)
    fire(0, 0)

    def pair(p2, carry):
        gA = 2 * p2
        fire(gA + 1, 1)
        wait_fire(0)

        @pl.when(p2 >= 1)
        def _():
            wait_out(0)

        extract_and_out(gA, 0)

        @pl.when(jnp.logical_and((p2 + 1) % (NBI // 2) == 0, p2 < NPAIR - 1))
        def _():
            stage((p2 + 1) // (NBI // 2))

        @pl.when(p2 < NPAIR - 1)
        def _():
            fire(gA + 2, 0)

        wait_fire(1)

        @pl.when(p2 >= 1)
        def _():
            wait_out(1)

        extract_and_out(gA + 1, 1)
        return carry

    lax.fori_loop(0, NPAIR, pair, 0)
    wait_out(0)
    wait_out(1)


def kernel(images, transformations, T_cw, coordinates):
    coords2 = coordinates.reshape(3, N)
    R = T_cw[:, :3, :3]
    t = T_cw[:, :3, 3]
    cam = -jnp.einsum('ikl,ik->il', R, t)
    trb = transformations.astype(jnp.bfloat16).astype(jnp.float32)
    idx, ext = _tc_project(coords2, trb, cam)
    table = _tc_table(images)
    out = _sc_gather(table, idx.reshape(I * N // GCH, GCH), ext)
    return out.reshape(I // 2, 2, CO, 64, 64, 64)


# speedup vs baseline: 2.5304x; 1.1551x over previous
import functools

import jax
import jax.numpy as jnp
from jax import lax
from jax.experimental import pallas as pl
from jax.experimental.pallas import tpu as pltpu
from jax.experimental.pallas import tpu_sc as plsc

I, C, H, W = 8, 28, 512, 512
HW = H * W
CP = 32
CE = 5
CO = C + CE
N = 64 * 64 * 64
EPS = 1e-8

NC, NS, L = 2, 16, 16
NW = NC * NS
PPW = N // NW
BLK = 512
NBLK = PPW // BLK
GCH = 64
NG = BLK // GCH
BN = 2048
IH = 4


def _tc_project_body(coord_ref, tr_ref, cam_ref, idx_ref, ext_ref):
    x = coord_ref[0:1, :]
    y = coord_ref[1:2, :]
    z = coord_ref[2:3, :]
    bf = lambda a: a.astype(jnp.bfloat16).astype(jnp.float32)
    xb, yb, zb = bf(x), bf(y), bf(z)
    for i in range(I):
        t = lambda k, l: tr_ref[i, k, l]
        p0 = (t(0, 0) * xb + t(0, 1) * yb) + (t(0, 2) * zb + t(0, 3))
        p1 = (t(1, 0) * xb + t(1, 1) * yb) + (t(1, 2) * zb + t(1, 3))
        d = (t(2, 0) * xb + t(2, 1) * yb) + (t(2, 2) * zb + t(2, 3))
        u = p0 / (d + EPS)
        v = p1 / (d + EPS)
        valid = (d > 0) & (u >= 0) & (u <= W - 1) & (v >= 0) & (v <= H - 1)
        validf = valid.astype(jnp.float32)
        ui = jnp.clip(jnp.round(u), 0, W - 1).astype(jnp.int32)
        vi = jnp.clip(jnp.round(v), 0, H - 1).astype(jnp.int32)
        ui = jnp.clip(ui, 0, W - 1)
        vi = jnp.clip(vi, 0, H - 1)
        pos = (jax.lax.broadcasted_iota(jnp.int32, (1, BN), 1)
               + pl.program_id(0) * BN)
        ih = i * HW
        idx_ref[i:i + 1, :] = jnp.where(valid, ih + vi * W + ui, -1 - pos)
        dx = x - cam_ref[i, 0]
        dy = y - cam_ref[i, 1]
        dz = z - cam_ref[i, 2]
        inv = 1.0 / (jnp.sqrt(dx * dx + dy * dy + dz * dz) + EPS)
        ext_ref[i, 0:1, :] = d
        ext_ref[i, 1:2, :] = validf
        ext_ref[i, 2:3, :] = dx * inv
        ext_ref[i, 3:4, :] = dy * inv
        ext_ref[i, 4:5, :] = dz * inv


_tc_project = pl.pallas_call(
    _tc_project_body,
    grid=(N // BN,),
    in_specs=[
        pl.BlockSpec((3, BN), lambda j: (0, j)),
        pl.BlockSpec(memory_space=pltpu.SMEM),
        pl.BlockSpec(memory_space=pltpu.SMEM),
    ],
    out_specs=[
        pl.BlockSpec((I, BN), lambda j: (0, j)),
        pl.BlockSpec((I, CE, BN), lambda j: (0, 0, j)),
    ],
    out_shape=[
        jax.ShapeDtypeStruct((I, N), jnp.int32),
        jax.ShapeDtypeStruct((I, CE, N), jnp.float32),
    ],
)


BH = 8


def _tc_table_body(img_ref, tab_ref):
    x = img_ref[0]
    x = x.reshape(C, BH * W)
    xp = jnp.concatenate([x, jnp.zeros((CP - C, BH * W), x.dtype)], axis=0)
    tab_ref[...] = xp.T


_tc_table = pl.pallas_call(
    _tc_table_body,
    grid=(I, H // BH),
    in_specs=[pl.BlockSpec((1, C, BH, W), lambda i, h: (i, 0, h, 0))],
    out_specs=pl.BlockSpec((BH * W, CP), lambda i, h: (i * (H // BH) + h, 0)),
    out_shape=jax.ShapeDtypeStruct((I * HW, CP), jnp.float32),
)


_sc_mesh = plsc.VectorSubcoreMesh(core_axis_name="c", subcore_axis_name="s",
                                  num_cores=NC, num_subcores=NS)


NBI = PPW // BLK
RPI = PPW // GCH
NPAIR = I * NBI // 2


@functools.partial(
    pl.kernel,
    out_type=jax.ShapeDtypeStruct((I, CO, N), jnp.float32),
    mesh=_sc_mesh,
    compiler_params=pltpu.CompilerParams(needs_layout_passes=False,
                                         use_tc_tiling_on_sc=False),
    scratch_types=[
        pltpu.VMEM((2 * PPW // L, L), jnp.int32),
        pltpu.VMEM((BLK,), jnp.int32),
        pltpu.VMEM((BLK,), jnp.int32),
        pltpu.VMEM((BLK,), jnp.int32),
        pltpu.VMEM((BLK,), jnp.int32),
        pltpu.SMEM((2,), jnp.int32),
        pltpu.VMEM((BLK, CP), jnp.float32),
        pltpu.VMEM((BLK, CP), jnp.float32),
        pltpu.VMEM((CO, BLK), jnp.float32),
        pltpu.VMEM((CO, BLK), jnp.float32),
        pltpu.VMEM((CE, BLK), jnp.float32),
        pltpu.VMEM((CE, BLK), jnp.float32),
        pltpu.SemaphoreType.DMA,
        pltpu.SemaphoreType.DMA,
        pltpu.SemaphoreType.DMA,
        pltpu.SemaphoreType.DMA,
        pltpu.SemaphoreType.DMA,
        pltpu.SemaphoreType.DMA,
    ],
)
def _sc_gather(table_hbm, idx_hbm, ext_hbm, out_hbm, idxv,
               gb0, gb1, pb0, pb1, cnts,
               rows0, rows1, cb0, cb1, eb0, eb1,
               gs0, gs1, es0, es1, os0, os1):
    wid = lax.axis_index("s") * NC + lax.axis_index("c")
    iota = lax.iota(jnp.int32, L)
    rows = (rows0, rows1)
    gbs = (gb0, gb1)
    pbs = (pb0, pb1)
    cbs = (cb0, cb1)
    ebs = (eb0, eb1)
    gss = (gs0, gs1)
    ess = (es0, es1)
    oss = (os0, os1)

    GR = PPW // L

    def stage(ib):
        row0 = ib * (N // L) + wid * GR
        pltpu.sync_copy(idx_hbm.at[pl.ds(row0, GR)],
                        idxv.at[pl.ds((ib % 2) * GR, GR)])

    def fire(g, p):
        ib = g // NBI
        lb = g - ib * NBI
        gbase = (ib % 2) * GR + lb * (BLK // L)

        def cg(r, carry):
            iv = idxv[gbase + r, :]
            m = iv >= 0
            vint = jnp.where(m, 1, 0).astype(jnp.int32)
            pos = carry + plsc.cumsum(vint) - vint
            plsc.store_scatter(gbs[p], [pos], iv, mask=m)
            pbs[p][pl.ds(r * L, L)] = pos
            return carry + plsc.all_reduce_population_count(m)

        cntv = lax.fori_loop(0, BLK // L, cg, jnp.zeros((L,), jnp.int32))
        cnt = lax.reduce_max(cntv, axes=(0,))
        cnts[p] = cnt
        for j in range(NG):
            @pl.when(cnt > j * GCH)
            def _():
                pltpu.async_copy(table_hbm.at[gbs[p].at[pl.ds(j * GCH, GCH)]],
                                 rows[p].at[pl.ds(j * GCH, GCH)], gss[p])
        nb = wid * PPW + lb * BLK
        pltpu.async_copy(ext_hbm.at[ib, :, pl.ds(nb, BLK)], ebs[p], ess[p])

    def wait_fire(p):
        cnt = cnts[p]
        for j in range(NG):
            @pl.when(cnt > j * GCH)
            def _():
                pltpu.make_async_copy(
                    table_hbm.at[gbs[p].at[pl.ds(j * GCH, GCH)]],
                    rows[p].at[pl.ds(j * GCH, GCH)], gss[p]).wait()
        pltpu.make_async_copy(ext_hbm.at[0, :, pl.ds(0, BLK)], ebs[p],
                              ess[p]).wait()

    def wait_out(p):
        pltpu.make_async_copy(cbs[p], out_hbm.at[0, :, pl.ds(0, BLK)],
                              oss[p]).wait()

    def extract_and_out(g, p):
        ib = g // NBI
        lb = g - ib * NBI
        nb = wid * PPW + lb * BLK

        def grp(r, _):
            r0 = r * L
            ridx = jnp.minimum(pbs[p][pl.ds(r0, L)], BLK - 1)
            vf = ebs[p][1, pl.ds(r0, L)]
            for c in range(C):
                cidx = jnp.full((L,), c, jnp.int32)
                val = plsc.load_gather(rows[p], [ridx, cidx])
                cbs[p][c, pl.ds(r0, L)] = val * vf
            for e in range(CE):
                cbs[p][C + e, pl.ds(r0, L)] = ebs[p][e, pl.ds(r0, L)]
            return 0

        lax.fori_loop(0, BLK // L, grp, 0)
        pltpu.async_copy(cbs[p], out_hbm.at[ib, :, pl.ds(nb, BLK)], oss[p])

    def prefill(r, carry):
        z = jnp.zeros((L,), jnp.int32)
        gb0[pl.ds(r * L, L)] = z
        gb1[pl.ds(r * L, L)] = z
        return carry

    lax.fori_loop(0, BLK // L, prefill, 0)
    stage(0)
    fire(0, 0)

    def pair(p2, carry):
        gA = 2 * p2
        fire(gA + 1, 1)
        wait_fire(0)

        @pl.when(p2 >= 1)
        def _():
            wait_out(0)

        extract_and_out(gA, 0)

        @pl.when(jnp.logical_and((p2 + 1) % (NBI // 2) == 0, p2 < NPAIR - 1))
        def _():
            stage((p2 + 1) // (NBI // 2))

        @pl.when(p2 < NPAIR - 1)
        def _():
            fire(gA + 2, 0)

        wait_fire(1)

        @pl.when(p2 >= 1)
        def _():
            wait_out(1)

        extract_and_out(gA + 1, 1)
        return carry

    lax.fori_loop(0, NPAIR, pair, 0)
    wait_out(0)
    wait_out(1)


def kernel(images, transformations, T_cw, coordinates):
    coords2 = coordinates.reshape(3, N)
    R = T_cw[:, :3, :3]
    t = T_cw[:, :3, 3]
    cam = -jnp.einsum('ikl,ik->il', R, t)
    trb = transformations.astype(jnp.bfloat16).astype(jnp.float32)
    idx, ext = _tc_project(coords2, trb, cam)
    table = _tc_table(images)
    out = _sc_gather(table, idx.reshape(I * N // L, L), ext)
    return out.reshape(I // 2, 2, CO, 64, 64, 64)


# speedup vs baseline: 2.6252x; 1.0374x over previous
import functools

import jax
import jax.numpy as jnp
from jax import lax
from jax.experimental import pallas as pl
from jax.experimental.pallas import tpu as pltpu
from jax.experimental.pallas import tpu_sc as plsc

I, C, H, W = 8, 28, 512, 512
HW = H * W
CP = 32
CE = 5
CO = C + CE
N = 64 * 64 * 64
EPS = 1e-8

NC, NS, L = 2, 16, 16
NW = NC * NS
PPW = N // NW
BLK = 512
NBLK = PPW // BLK
GCH = 64
NG = BLK // GCH
BN = 2048
IH = 4


def _tc_project_body(coord_ref, tr_ref, cam_ref, idx_ref, ext_ref):
    x = coord_ref[0:1, :]
    y = coord_ref[1:2, :]
    z = coord_ref[2:3, :]
    bf = lambda a: a.astype(jnp.bfloat16).astype(jnp.float32)
    xb, yb, zb = bf(x), bf(y), bf(z)
    for i in range(I):
        t = lambda k, l: tr_ref[i, k, l]
        p0 = (t(0, 0) * xb + t(0, 1) * yb) + (t(0, 2) * zb + t(0, 3))
        p1 = (t(1, 0) * xb + t(1, 1) * yb) + (t(1, 2) * zb + t(1, 3))
        d = (t(2, 0) * xb + t(2, 1) * yb) + (t(2, 2) * zb + t(2, 3))
        u = p0 / (d + EPS)
        v = p1 / (d + EPS)
        valid = (d > 0) & (u >= 0) & (u <= W - 1) & (v >= 0) & (v <= H - 1)
        validf = valid.astype(jnp.float32)
        ui = jnp.clip(jnp.round(u), 0, W - 1).astype(jnp.int32)
        vi = jnp.clip(jnp.round(v), 0, H - 1).astype(jnp.int32)
        ui = jnp.clip(ui, 0, W - 1)
        vi = jnp.clip(vi, 0, H - 1)
        pos = (jax.lax.broadcasted_iota(jnp.int32, (1, BN), 1)
               + pl.program_id(0) * BN)
        ih = i * HW
        idx_ref[i:i + 1, :] = jnp.where(valid, ih + vi * W + ui, -1 - pos)
        dx = x - cam_ref[i, 0]
        dy = y - cam_ref[i, 1]
        dz = z - cam_ref[i, 2]
        inv = 1.0 / (jnp.sqrt(dx * dx + dy * dy + dz * dz) + EPS)
        ext_ref[i, 0:1, :] = d
        ext_ref[i, 1:2, :] = validf
        ext_ref[i, 2:3, :] = dx * inv
        ext_ref[i, 3:4, :] = dy * inv
        ext_ref[i, 4:5, :] = dz * inv


_tc_project = pl.pallas_call(
    _tc_project_body,
    grid=(N // BN,),
    in_specs=[
        pl.BlockSpec((3, BN), lambda j: (0, j)),
        pl.BlockSpec(memory_space=pltpu.SMEM),
        pl.BlockSpec(memory_space=pltpu.SMEM),
    ],
    out_specs=[
        pl.BlockSpec((I, BN), lambda j: (0, j)),
        pl.BlockSpec((I, CE, BN), lambda j: (0, 0, j)),
    ],
    out_shape=[
        jax.ShapeDtypeStruct((I, N), jnp.int32),
        jax.ShapeDtypeStruct((I, CE, N), jnp.float32),
    ],
)


BH = 16


def _tc_table_body(img_ref, tab_ref):
    x = img_ref[0]
    x = x.reshape(C, BH * W)
    xp = jnp.concatenate([x, jnp.zeros((CP - C, BH * W), x.dtype)], axis=0)
    tab_ref[...] = xp.T


_tc_table = pl.pallas_call(
    _tc_table_body,
    grid=(I, H // BH),
    in_specs=[pl.BlockSpec((1, C, BH, W), lambda i, h: (i, 0, h, 0))],
    out_specs=pl.BlockSpec((BH * W, CP), lambda i, h: (i * (H // BH) + h, 0)),
    out_shape=jax.ShapeDtypeStruct((I * HW, CP), jnp.float32),
)


_sc_mesh = plsc.VectorSubcoreMesh(core_axis_name="c", subcore_axis_name="s",
                                  num_cores=NC, num_subcores=NS)


NBI = PPW // BLK
RPI = PPW // GCH
NPAIR = I * NBI // 2


@functools.partial(
    pl.kernel,
    out_type=jax.ShapeDtypeStruct((I, CO, N), jnp.float32),
    mesh=_sc_mesh,
    compiler_params=pltpu.CompilerParams(needs_layout_passes=False,
                                         use_tc_tiling_on_sc=False),
    scratch_types=[
        pltpu.VMEM((2 * PPW // L, L), jnp.int32),
        pltpu.VMEM((BLK,), jnp.int32),
        pltpu.VMEM((BLK,), jnp.int32),
        pltpu.VMEM((BLK,), jnp.int32),
        pltpu.VMEM((BLK,), jnp.int32),
        pltpu.SMEM((2,), jnp.int32),
        pltpu.VMEM((BLK, CP), jnp.float32),
        pltpu.VMEM((BLK, CP), jnp.float32),
        pltpu.VMEM((CO, BLK), jnp.float32),
        pltpu.VMEM((CO, BLK), jnp.float32),
        pltpu.VMEM((CE, BLK), jnp.float32),
        pltpu.VMEM((CE, BLK), jnp.float32),
        pltpu.SemaphoreType.DMA,
        pltpu.SemaphoreType.DMA,
        pltpu.SemaphoreType.DMA,
        pltpu.SemaphoreType.DMA,
        pltpu.SemaphoreType.DMA,
        pltpu.SemaphoreType.DMA,
    ],
)
def _sc_gather(table_hbm, idx_hbm, ext_hbm, out_hbm, idxv,
               gb0, gb1, pb0, pb1, cnts,
               rows0, rows1, cb0, cb1, eb0, eb1,
               gs0, gs1, es0, es1, os0, os1):
    wid = lax.axis_index("s") * NC + lax.axis_index("c")
    iota = lax.iota(jnp.int32, L)
    rows = (rows0, rows1)
    gbs = (gb0, gb1)
    pbs = (pb0, pb1)
    cbs = (cb0, cb1)
    ebs = (eb0, eb1)
    gss = (gs0, gs1)
    ess = (es0, es1)
    oss = (os0, os1)

    GR = PPW // L

    def stage(ib):
        row0 = ib * (N // L) + wid * GR
        pltpu.sync_copy(idx_hbm.at[pl.ds(row0, GR)],
                        idxv.at[pl.ds((ib % 2) * GR, GR)])

    def fire(g, p):
        ib = g // NBI
        lb = g - ib * NBI
        gbase = (ib % 2) * GR + lb * (BLK // L)

        def cg(r, carry):
            iv = idxv[gbase + r, :]
            m = iv >= 0
            vint = jnp.where(m, 1, 0).astype(jnp.int32)
            pos = carry + plsc.cumsum(vint) - vint
            plsc.store_scatter(gbs[p], [pos], iv, mask=m)
            pbs[p][pl.ds(r * L, L)] = pos
            return carry + plsc.all_reduce_population_count(m)

        cntv = lax.fori_loop(0, BLK // L, cg, jnp.zeros((L,), jnp.int32))
        cnt = lax.reduce_max(cntv, axes=(0,))
        cnts[p] = cnt
        for j in range(NG):
            @pl.when(cnt > j * GCH)
            def _():
                pltpu.async_copy(table_hbm.at[gbs[p].at[pl.ds(j * GCH, GCH)]],
                                 rows[p].at[pl.ds(j * GCH, GCH)], gss[p])
        nb = wid * PPW + lb * BLK
        pltpu.async_copy(ext_hbm.at[ib, :, pl.ds(nb, BLK)], ebs[p], ess[p])

    def wait_fire(p):
        cnt = cnts[p]
        for j in range(NG):
            @pl.when(cnt > j * GCH)
            def _():
                pltpu.make_async_copy(
                    table_hbm.at[gbs[p].at[pl.ds(j * GCH, GCH)]],
                    rows[p].at[pl.ds(j * GCH, GCH)], gss[p]).wait()
        pltpu.make_async_copy(ext_hbm.at[0, :, pl.ds(0, BLK)], ebs[p],
                              ess[p]).wait()

    def wait_out(p):
        pltpu.make_async_copy(cbs[p], out_hbm.at[0, :, pl.ds(0, BLK)],
                              oss[p]).wait()

    def extract_and_out(g, p):
        ib = g // NBI
        lb = g - ib * NBI
        nb = wid * PPW + lb * BLK

        def grp(r, _):
            r0 = r * L
            ridx = jnp.minimum(pbs[p][pl.ds(r0, L)], BLK - 1)
            vf = ebs[p][1, pl.ds(r0, L)]
            for c in range(C):
                cidx = jnp.full((L,), c, jnp.int32)
                val = plsc.load_gather(rows[p], [ridx, cidx])
                cbs[p][c, pl.ds(r0, L)] = val * vf
            for e in range(CE):
                cbs[p][C + e, pl.ds(r0, L)] = ebs[p][e, pl.ds(r0, L)]
            return 0

        lax.fori_loop(0, BLK // L, grp, 0)
        pltpu.async_copy(cbs[p], out_hbm.at[ib, :, pl.ds(nb, BLK)], oss[p])

    def prefill(r, carry):
        z = jnp.zeros((L,), jnp.int32)
        gb0[pl.ds(r * L, L)] = z
        gb1[pl.ds(r * L, L)] = z
        return carry

    lax.fori_loop(0, BLK // L, prefill, 0)
    stage(0)
    fire(0, 0)

    def pair(p2, carry):
        gA = 2 * p2
        fire(gA + 1, 1)
        wait_fire(0)

        @pl.when(p2 >= 1)
        def _():
            wait_out(0)

        extract_and_out(gA, 0)

        @pl.when(jnp.logical_and((p2 + 1) % (NBI // 2) == 0, p2 < NPAIR - 1))
        def _():
            stage((p2 + 1) // (NBI // 2))

        @pl.when(p2 < NPAIR - 1)
        def _():
            fire(gA + 2, 0)

        wait_fire(1)

        @pl.when(p2 >= 1)
        def _():
            wait_out(1)

        extract_and_out(gA + 1, 1)
        return carry

    lax.fori_loop(0, NPAIR, pair, 0)
    wait_out(0)
    wait_out(1)


def kernel(images, transformations, T_cw, coordinates):
    coords2 = coordinates.reshape(3, N)
    R = T_cw[:, :3, :3]
    t = T_cw[:, :3, 3]
    cam = -jnp.einsum('ikl,ik->il', R, t)
    trb = transformations.astype(jnp.bfloat16).astype(jnp.float32)
    idx, ext = _tc_project(coords2, trb, cam)
    table = _tc_table(images)
    out = _sc_gather(table, idx.reshape(I * N // L, L), ext)
    return out.reshape(I // 2, 2, CO, 64, 64, 64)
